# Initial kernel scaffold; baseline (speedup 1.0000x reference)
import jax
import jax.numpy as jnp
from jax.experimental import pallas as pl


def kernel(x):
    raise NotImplementedError("write your pallas kernel here")


# speedup vs baseline: 13.0321x; 13.0321x over previous
import functools

import jax
import jax.numpy as jnp
from jax.experimental import pallas as pl
from jax.experimental.pallas import tpu as pltpu

_N = 4096
_ROWS = 32 * 64
_BLOCK_ROWS = 256

_K_LO = 81
_F_LO = 0.02 * (_N - 1) - _K_LO
_K_HI = 4013
_F_HI = 0.98 * (_N - 1) - _K_HI

_IMIN = -(2**31)
_IMAX = 2**31 - 1


def _to_key(x):
    i = jax.lax.bitcast_convert_type(x, jnp.int32)
    return i ^ ((i >> 31) & jnp.int32(0x7FFFFFFF))


def _from_key(k):
    i = k ^ ((k >> 31) & jnp.int32(0x7FFFFFFF))
    return jax.lax.bitcast_convert_type(i, jnp.float32)


def _percentile_normalize_kernel(x_ref, o_ref, key_ref):
    xb = x_ref[...]
    key = _to_key(xb)
    key_ref[...] = key

    def count_le(t):
        return jnp.sum((key_ref[...] <= t).astype(jnp.int32), axis=1,
                       keepdims=True)

    r = _BLOCK_ROWS
    lo1 = jnp.full((r, 1), _IMIN, jnp.int32)
    hi1 = jnp.full((r, 1), _IMAX, jnp.int32)
    lo2 = jnp.full((r, 1), _IMIN, jnp.int32)
    hi2 = jnp.full((r, 1), _IMAX, jnp.int32)

    def body(_, carry):
        lo1, hi1, lo2, hi2 = carry
        mid1 = (lo1 & hi1) + ((lo1 ^ hi1) >> 1)
        mid2 = (lo2 & hi2) + ((lo2 ^ hi2) >> 1)
        k = key_ref[...]
        c1 = jnp.sum((k <= mid1).astype(jnp.int32), axis=1, keepdims=True)
        c2 = jnp.sum((k <= mid2).astype(jnp.int32), axis=1, keepdims=True)
        p1 = c1 >= _K_LO + 1
        p2 = c2 >= _K_HI + 1
        lo1 = jnp.where(p1, lo1, mid1 + 1)
        hi1 = jnp.where(p1, mid1, hi1)
        lo2 = jnp.where(p2, lo2, mid2 + 1)
        hi2 = jnp.where(p2, mid2, hi2)
        return lo1, hi1, lo2, hi2

    lo1, hi1, lo2, hi2 = jax.lax.fori_loop(
        0, 32, body, (lo1, hi1, lo2, hi2))
    kA = lo1
    kB = lo2

    k = key_ref[...]
    le_a = k <= kA
    cntA = jnp.sum(le_a.astype(jnp.int32), axis=1, keepdims=True)
    min_gtA = jnp.min(jnp.where(le_a, jnp.int32(_IMAX), k), axis=1,
                      keepdims=True)
    kA1 = jnp.where(cntA >= _K_LO + 2, kA, min_gtA)

    le_b = k <= kB
    cntB = jnp.sum(le_b.astype(jnp.int32), axis=1, keepdims=True)
    min_gtB = jnp.min(jnp.where(le_b, jnp.int32(_IMAX), k), axis=1,
                      keepdims=True)
    kB1 = jnp.where(cntB >= _K_HI + 2, kB, min_gtB)

    vA = _from_key(kA)
    vA1 = _from_key(kA1)
    vB = _from_key(kB)
    vB1 = _from_key(kB1)

    lower = vA + jnp.float32(_F_LO) * (vA1 - vA)
    upper = vB + jnp.float32(_F_HI) * (vB1 - vB)

    o_ref[...] = (xb - lower) / (upper - lower)


@jax.jit
def kernel(x):
    b, c, n = x.shape
    xr = x.reshape(b * c, n)
    out = pl.pallas_call(
        _percentile_normalize_kernel,
        grid=(_ROWS // _BLOCK_ROWS,),
        in_specs=[pl.BlockSpec((_BLOCK_ROWS, _N), lambda i: (i, 0))],
        out_specs=pl.BlockSpec((_BLOCK_ROWS, _N), lambda i: (i, 0)),
        out_shape=jax.ShapeDtypeStruct((b * c, n), jnp.float32),
        scratch_shapes=[pltpu.VMEM((_BLOCK_ROWS, _N), jnp.int32)],
    )(xr)
    return out.reshape(b, c, n)
